# baseline (speedup 1.0000x reference)
import jax
import jax.numpy as jnp
from jax.experimental import pallas as pl
from jax.experimental.pallas import tpu as pltpu

K = 64
N_ROWS = 2048
D_IN = 768
D_SAE = 32768


ENC_CHUNK = 4096
ENC_ROWS = 256


def _encode_body(x_ref, wenc_ref, benc_ref, bdec_ref, pre_ref, preb_ref):
    sae_in = x_ref[...] - bdec_ref[...]
    acc = jnp.dot(sae_in, wenc_ref[...], preferred_element_type=jnp.float32)
    acc = jnp.maximum(acc + benc_ref[...], 0.0)
    pre_ref[...] = acc
    preb_ref[...] = acc.astype(jnp.bfloat16)


def _encode(x, w_enc, b_enc, b_dec):
    n_chunks = D_SAE // ENC_CHUNK
    n_rt = N_ROWS // ENC_ROWS
    return pl.pallas_call(
        _encode_body,
        grid=(n_chunks, n_rt),
        in_specs=[
            pl.BlockSpec((ENC_ROWS, D_IN), lambda c, r: (r, 0)),
            pl.BlockSpec((D_IN, ENC_CHUNK), lambda c, r: (0, c)),
            pl.BlockSpec((1, ENC_CHUNK), lambda c, r: (0, c)),
            pl.BlockSpec((1, D_IN), lambda c, r: (0, 0)),
        ],
        out_specs=[
            pl.BlockSpec((ENC_ROWS, ENC_CHUNK), lambda c, r: (r, c)),
            pl.BlockSpec((ENC_ROWS, ENC_CHUNK), lambda c, r: (r, c)),
        ],
        out_shape=[
            jax.ShapeDtypeStruct((N_ROWS, D_SAE), jnp.float32),
            jax.ShapeDtypeStruct((N_ROWS, D_SAE), jnp.bfloat16),
        ],
        compiler_params=pltpu.CompilerParams(
            dimension_semantics=("arbitrary", "parallel"),
        ),
    )(x, w_enc, b_enc, b_dec)


THR_ROWS = 64


def _select_body(pre_ref, preb_ref, z_ref):
    def bstep(_, carry):
        lo, hi = carry
        mid = lo + (hi - lo + 1) // 2
        midb = pltpu.bitcast(mid << 16, jnp.float32).astype(jnp.bfloat16)
        mask = (preb_ref[...] >= midb).astype(jnp.float32)
        cnt = jnp.sum(mask, axis=1, keepdims=True)
        ge = cnt >= float(K)
        return jnp.where(ge, mid, lo), jnp.where(ge, hi, mid - 1)

    blo0 = jnp.zeros((THR_ROWS, 1), jnp.int32)
    bhi0 = jnp.full((THR_ROWS, 1), 0x7F80, jnp.int32)
    h, _ = jax.lax.fori_loop(0, 15, bstep, (blo0, bhi0))

    lo0 = jnp.maximum((h << 16) - 0x8000, 0)
    hi0 = (h << 16) + 0x8001

    def step(_, carry):
        lo, hi = carry
        mid = lo + (hi - lo + 1) // 2
        midf = pltpu.bitcast(mid, jnp.float32)
        cnt = jnp.sum(
            (pre_ref[...] >= midf).astype(jnp.float32), axis=1, keepdims=True
        )
        ge = cnt >= float(K)
        return jnp.where(ge, mid, lo), jnp.where(ge, hi, mid - 1)

    lo, _ = jax.lax.fori_loop(0, 17, step, (lo0, hi0))
    thr = pltpu.bitcast(lo, jnp.float32)
    pre2 = pre_ref[...]
    z_ref[...] = jnp.where(pre2 >= thr, pre2, 0.0).astype(jnp.bfloat16)


def _select(pre, preb):
    n_rt = N_ROWS // THR_ROWS
    return pl.pallas_call(
        _select_body,
        grid=(n_rt,),
        in_specs=[
            pl.BlockSpec((THR_ROWS, D_SAE), lambda r: (r, 0)),
            pl.BlockSpec((THR_ROWS, D_SAE), lambda r: (r, 0)),
        ],
        out_specs=pl.BlockSpec((THR_ROWS, D_SAE), lambda r: (r, 0)),
        out_shape=jax.ShapeDtypeStruct((N_ROWS, D_SAE), jnp.bfloat16),
        compiler_params=pltpu.CompilerParams(
            dimension_semantics=("parallel",),
        ),
    )(pre, preb)


DEC_ROWS = 512
DEC_CHUNK = 4096


def _decode_body(z_ref, wdec_ref, bdec_ref, out_ref):
    c = pl.program_id(1)
    part = jnp.dot(z_ref[...], wdec_ref[...], preferred_element_type=jnp.float32)

    @pl.when(c == 0)
    def _():
        out_ref[...] = part + bdec_ref[...]

    @pl.when(c > 0)
    def _():
        out_ref[...] += part


def _decode(z, w_dec_bf16, b_dec):
    n_rt = N_ROWS // DEC_ROWS
    n_chunks = D_SAE // DEC_CHUNK
    return pl.pallas_call(
        _decode_body,
        grid=(n_rt, n_chunks),
        in_specs=[
            pl.BlockSpec((DEC_ROWS, DEC_CHUNK), lambda r, c: (r, c)),
            pl.BlockSpec((DEC_CHUNK, D_IN), lambda r, c: (c, 0)),
            pl.BlockSpec((1, D_IN), lambda r, c: (0, 0)),
        ],
        out_specs=pl.BlockSpec((DEC_ROWS, D_IN), lambda r, c: (r, 0)),
        out_shape=jax.ShapeDtypeStruct((N_ROWS, D_IN), jnp.float32),
        compiler_params=pltpu.CompilerParams(
            dimension_semantics=("parallel", "arbitrary"),
        ),
    )(z, w_dec_bf16, b_dec)


@jax.jit
def _run(x, w_enc, b_enc, w_dec, b_dec):
    x2 = x.reshape(-1, D_IN)
    pre, preb = _encode(x2, w_enc, b_enc.reshape(1, -1), b_dec.reshape(1, -1))
    z = _select(pre, preb)
    out = _decode(z, w_dec.astype(jnp.bfloat16), b_dec.reshape(1, -1))
    return out.reshape(x.shape[:-1] + (D_IN,))


def kernel(x, W_enc, b_enc, W_dec, b_dec):
    return _run(x, W_enc, b_enc, W_dec, b_dec)


# speedup vs baseline: 1.2968x; 1.2968x over previous
import jax
import jax.numpy as jnp
from jax.experimental import pallas as pl
from jax.experimental.pallas import tpu as pltpu

K = 64
N_ROWS = 2048
D_IN = 768
D_SAE = 32768


ENC_CHUNK = 4096
ENC_ROWS = 256


def _encode_body(x_ref, wenc_ref, benc_ref, bdec_ref, pre_ref):
    sae_in = x_ref[...] - bdec_ref[...]
    acc = jnp.dot(sae_in, wenc_ref[...], preferred_element_type=jnp.float32)
    pre_ref[...] = jnp.maximum(acc + benc_ref[...], 0.0)


def _encode(x, w_enc, b_enc, b_dec):
    n_chunks = D_SAE // ENC_CHUNK
    n_rt = N_ROWS // ENC_ROWS
    return pl.pallas_call(
        _encode_body,
        grid=(n_chunks, n_rt),
        in_specs=[
            pl.BlockSpec((ENC_ROWS, D_IN), lambda c, r: (r, 0)),
            pl.BlockSpec((D_IN, ENC_CHUNK), lambda c, r: (0, c)),
            pl.BlockSpec((1, ENC_CHUNK), lambda c, r: (0, c)),
            pl.BlockSpec((1, D_IN), lambda c, r: (0, 0)),
        ],
        out_specs=pl.BlockSpec((ENC_ROWS, ENC_CHUNK), lambda c, r: (r, c)),
        out_shape=jax.ShapeDtypeStruct((N_ROWS, D_SAE), jnp.float32),
        compiler_params=pltpu.CompilerParams(
            dimension_semantics=("arbitrary", "parallel"),
        ),
    )(x, w_enc, b_enc, b_dec)


THR_ROWS = 128


def _select_body(pre_ref, z_ref):
    def step(_, carry):
        lo, hi = carry
        mid = lo + (hi - lo + 1) // 2
        midf = pltpu.bitcast(mid, jnp.float32)
        cnt = jnp.sum(
            (pre_ref[...] >= midf).astype(jnp.float32), axis=1, keepdims=True
        )
        ge = cnt >= float(K)
        return jnp.where(ge, mid, lo), jnp.where(ge, hi, mid - 1)

    lo0 = jnp.zeros((THR_ROWS, 1), jnp.int32)
    hi0 = jnp.full((THR_ROWS, 1), 0x7F800000, jnp.int32)
    lo, _ = jax.lax.fori_loop(0, 31, step, (lo0, hi0))
    thr = pltpu.bitcast(lo, jnp.float32)
    pre2 = pre_ref[...]
    z_ref[...] = jnp.where(pre2 >= thr, pre2, 0.0).astype(jnp.bfloat16)


def _select(pre):
    n_rt = N_ROWS // THR_ROWS
    return pl.pallas_call(
        _select_body,
        grid=(n_rt,),
        in_specs=[pl.BlockSpec((THR_ROWS, D_SAE), lambda r: (r, 0))],
        out_specs=pl.BlockSpec((THR_ROWS, D_SAE), lambda r: (r, 0)),
        out_shape=jax.ShapeDtypeStruct((N_ROWS, D_SAE), jnp.bfloat16),
        compiler_params=pltpu.CompilerParams(
            dimension_semantics=("parallel",),
        ),
    )(pre)


DEC_ROWS = 1024
DEC_CHUNK = 4096


def _decode_body(z_ref, wdec_ref, bdec_ref, out_ref):
    c = pl.program_id(1)
    part = jnp.dot(z_ref[...], wdec_ref[...], preferred_element_type=jnp.float32)

    @pl.when(c == 0)
    def _():
        out_ref[...] = part + bdec_ref[...]

    @pl.when(c > 0)
    def _():
        out_ref[...] += part


def _decode(z, w_dec_bf16, b_dec):
    n_rt = N_ROWS // DEC_ROWS
    n_chunks = D_SAE // DEC_CHUNK
    return pl.pallas_call(
        _decode_body,
        grid=(n_rt, n_chunks),
        in_specs=[
            pl.BlockSpec((DEC_ROWS, DEC_CHUNK), lambda r, c: (r, c)),
            pl.BlockSpec((DEC_CHUNK, D_IN), lambda r, c: (c, 0)),
            pl.BlockSpec((1, D_IN), lambda r, c: (0, 0)),
        ],
        out_specs=pl.BlockSpec((DEC_ROWS, D_IN), lambda r, c: (r, 0)),
        out_shape=jax.ShapeDtypeStruct((N_ROWS, D_IN), jnp.float32),
        compiler_params=pltpu.CompilerParams(
            dimension_semantics=("parallel", "arbitrary"),
        ),
    )(z, w_dec_bf16, b_dec)


@jax.jit
def _run(x, w_enc, b_enc, w_dec, b_dec):
    x2 = x.reshape(-1, D_IN)
    pre = _encode(x2, w_enc, b_enc.reshape(1, -1), b_dec.reshape(1, -1))
    z = _select(pre)
    out = _decode(z, w_dec.astype(jnp.bfloat16), b_dec.reshape(1, -1))
    return out.reshape(x.shape[:-1] + (D_IN,))


def kernel(x, W_enc, b_enc, W_dec, b_dec):
    return _run(x, W_enc, b_enc, W_dec, b_dec)


# speedup vs baseline: 1.3040x; 1.0056x over previous
import jax
import jax.numpy as jnp
from jax.experimental import pallas as pl
from jax.experimental.pallas import tpu as pltpu

K = 64
N_ROWS = 2048
D_IN = 768
D_SAE = 32768


ENC_CHUNK = 4096
ENC_ROWS = 512


def _encode_body(x_ref, wenc_ref, benc_ref, bdec_ref, pre_ref):
    sae_in = x_ref[...] - bdec_ref[...]
    acc = jnp.dot(sae_in, wenc_ref[...], preferred_element_type=jnp.float32)
    pre_ref[...] = jnp.maximum(acc + benc_ref[...], 0.0)


def _encode(x, w_enc, b_enc, b_dec):
    n_chunks = D_SAE // ENC_CHUNK
    n_rt = N_ROWS // ENC_ROWS
    return pl.pallas_call(
        _encode_body,
        grid=(n_chunks, n_rt),
        in_specs=[
            pl.BlockSpec((ENC_ROWS, D_IN), lambda c, r: (r, 0)),
            pl.BlockSpec((D_IN, ENC_CHUNK), lambda c, r: (0, c)),
            pl.BlockSpec((1, ENC_CHUNK), lambda c, r: (0, c)),
            pl.BlockSpec((1, D_IN), lambda c, r: (0, 0)),
        ],
        out_specs=pl.BlockSpec((ENC_ROWS, ENC_CHUNK), lambda c, r: (r, c)),
        out_shape=jax.ShapeDtypeStruct((N_ROWS, D_SAE), jnp.float32),
        compiler_params=pltpu.CompilerParams(
            dimension_semantics=("arbitrary", "parallel"),
        ),
    )(x, w_enc, b_enc, b_dec)


THR_ROWS = 128


def _select_body(pre_ref, z_ref):
    def step(_, carry):
        lo, hi = carry
        mid = lo + (hi - lo + 1) // 2
        midf = pltpu.bitcast(mid, jnp.float32)
        cnt = jnp.sum(
            (pre_ref[...] >= midf).astype(jnp.float32), axis=1, keepdims=True
        )
        ge = cnt >= float(K)
        return jnp.where(ge, mid, lo), jnp.where(ge, hi, mid - 1)

    lo0 = jnp.zeros((THR_ROWS, 1), jnp.int32)
    hi0 = jnp.full((THR_ROWS, 1), 0x7F800000, jnp.int32)
    lo, _ = jax.lax.fori_loop(0, 31, step, (lo0, hi0))
    thr = pltpu.bitcast(lo, jnp.float32)
    pre2 = pre_ref[...]
    z_ref[...] = jnp.where(pre2 >= thr, pre2, 0.0).astype(jnp.bfloat16)


def _select(pre):
    n_rt = N_ROWS // THR_ROWS
    return pl.pallas_call(
        _select_body,
        grid=(n_rt,),
        in_specs=[pl.BlockSpec((THR_ROWS, D_SAE), lambda r: (r, 0))],
        out_specs=pl.BlockSpec((THR_ROWS, D_SAE), lambda r: (r, 0)),
        out_shape=jax.ShapeDtypeStruct((N_ROWS, D_SAE), jnp.bfloat16),
        compiler_params=pltpu.CompilerParams(
            dimension_semantics=("parallel",),
        ),
    )(pre)


DEC_ROWS = 1024
DEC_CHUNK = 4096


def _decode_body(z_ref, wdec_ref, bdec_ref, out_ref):
    c = pl.program_id(1)
    part = jnp.dot(z_ref[...], wdec_ref[...], preferred_element_type=jnp.float32)

    @pl.when(c == 0)
    def _():
        out_ref[...] = part + bdec_ref[...]

    @pl.when(c > 0)
    def _():
        out_ref[...] += part


def _decode(z, w_dec_bf16, b_dec):
    n_rt = N_ROWS // DEC_ROWS
    n_chunks = D_SAE // DEC_CHUNK
    return pl.pallas_call(
        _decode_body,
        grid=(n_rt, n_chunks),
        in_specs=[
            pl.BlockSpec((DEC_ROWS, DEC_CHUNK), lambda r, c: (r, c)),
            pl.BlockSpec((DEC_CHUNK, D_IN), lambda r, c: (c, 0)),
            pl.BlockSpec((1, D_IN), lambda r, c: (0, 0)),
        ],
        out_specs=pl.BlockSpec((DEC_ROWS, D_IN), lambda r, c: (r, 0)),
        out_shape=jax.ShapeDtypeStruct((N_ROWS, D_IN), jnp.float32),
        compiler_params=pltpu.CompilerParams(
            dimension_semantics=("parallel", "arbitrary"),
        ),
    )(z, w_dec_bf16, b_dec)


@jax.jit
def _run(x, w_enc, b_enc, w_dec, b_dec):
    x2 = x.reshape(-1, D_IN)
    pre = _encode(x2, w_enc, b_enc.reshape(1, -1), b_dec.reshape(1, -1))
    z = _select(pre)
    out = _decode(z, w_dec.astype(jnp.bfloat16), b_dec.reshape(1, -1))
    return out.reshape(x.shape[:-1] + (D_IN,))


def kernel(x, W_enc, b_enc, W_dec, b_dec):
    return _run(x, W_enc, b_enc, W_dec, b_dec)
